# baseline (speedup 1.0000x reference)
import functools

import jax
import jax.numpy as jnp
from jax import lax
from jax.experimental import pallas as pl
from jax.experimental.pallas import tpu as pltpu
from jax.experimental.pallas import tpu_sc as plsc

_NODE_FEATS = 128
_STAT_MEDIAN = 0.0
_STAT_SCALE = 2.0 / (1.0 - (-1.0))
_N_NODES = 10000
_N_EDGES = 320000

_LANES = 16
_NW = 32
_E_PER_W = _N_EDGES // _NW
_CHUNK = 40
_CH_PER_W = _E_PER_W // _CHUNK
_NB = 4
_IDX_BLK = 50
_N_BLKS = _CH_PER_W // _IDX_BLK
_BLK_I = _IDX_BLK * 2 * _CHUNK

_mesh = plsc.VectorSubcoreMesh(core_axis_name="c", subcore_axis_name="s")


@functools.partial(
    pl.kernel,
    mesh=_mesh,
    out_type=jax.ShapeDtypeStruct((_N_EDGES, _NODE_FEATS), jnp.float32),
    scratch_types=[
        pltpu.VMEM_SHARED((_N_NODES, _NODE_FEATS), jnp.float32),
        pltpu.VMEM((2 * _BLK_I,), jnp.int32),
        pltpu.VMEM((_NB, 2 * _CHUNK, _NODE_FEATS), jnp.float32),
        pltpu.SemaphoreType.DMA((_NB,)),
        pltpu.SemaphoreType.DMA((_NB,)),
        pltpu.SemaphoreType.DMA((2,)),
    ],
)
def _edge_kernel(node_hbm, cidx_hbm, out_hbm,
                 table, cidx, rows, sem_g, sem_o, sem_i):
    wid = lax.axis_index("s") * 2 + lax.axis_index("c")
    ebase = wid * _E_PER_W
    ibase = wid * (2 * _E_PER_W)
    sid = lax.axis_index("s")

    rows_per_sub = 624
    tslice = pl.ds(sid * rows_per_sub, rows_per_sub)
    pltpu.async_copy(node_hbm.at[tslice], table.at[tslice], sem_i.at[0])
    rem = pl.ds(16 * rows_per_sub, _N_NODES - 16 * rows_per_sub)

    @pl.when(sid == 0)
    def _():
        pltpu.async_copy(node_hbm.at[rem], table.at[rem], sem_i.at[1])

    def fetch_idx(j, jbuf):
        pltpu.async_copy(cidx_hbm.at[pl.ds(ibase + j * _BLK_I, _BLK_I)],
                         cidx.at[pl.ds(jbuf * _BLK_I, _BLK_I)],
                         sem_i.at[jbuf])

    def wait_idx(j, jbuf):
        pltpu.make_async_copy(
            cidx_hbm.at[pl.ds(ibase + j * _BLK_I, _BLK_I)],
            cidx.at[pl.ds(jbuf * _BLK_I, _BLK_I)], sem_i.at[jbuf]).wait()

    pltpu.make_async_copy(node_hbm.at[tslice], table.at[tslice],
                          sem_i.at[0]).wait()

    @pl.when(sid == 0)
    def _():
        pltpu.make_async_copy(node_hbm.at[rem], table.at[rem],
                              sem_i.at[1]).wait()

    fetch_idx(0, 0)
    fetch_idx(1, 1)
    plsc.subcore_barrier()

    def idx_ref(i):
        off = ((i // _IDX_BLK) % 2) * _BLK_I + (i % _IDX_BLK) * (2 * _CHUNK)
        return cidx.at[pl.ds(off, 2 * _CHUNK)]

    def issue_gather(i, b):
        @pl.when(i % _IDX_BLK == 0)
        def _():
            wait_idx(i // _IDX_BLK, (i // _IDX_BLK) % 2)

        pltpu.async_copy(table.at[idx_ref(i)], rows.at[b], sem_g.at[b])

    def prefetch_idx(i):
        j2 = i // _IDX_BLK + 2

        @pl.when((i % _IDX_BLK == _IDX_BLK - 1) & (j2 < _N_BLKS))
        def _():
            fetch_idx(j2, j2 % 2)

    def wait_gather(i, b):
        pltpu.make_async_copy(table.at[idx_ref(i)], rows.at[b],
                              sem_g.at[b]).wait()

    def subtract(b):
        def sub_row(r, carry2):
            for r2 in range(2):
                for q in range(_NODE_FEATS // _LANES):
                    sl = pl.ds(q * _LANES, _LANES)
                    if _STAT_SCALE == 1.0:
                        plsc.addupdate(rows.at[b, 2 * r + r2, sl],
                                       -rows[b, _CHUNK + 2 * r + r2, sl])
                    else:
                        rows[b, 2 * r + r2, sl] = (
                            rows[b, 2 * r + r2, sl]
                            - rows[b, _CHUNK + 2 * r + r2, sl]) * _STAT_SCALE
            return carry2

        lax.fori_loop(0, _CHUNK // 2, sub_row, 0)

    def out_slice(i):
        return out_hbm.at[pl.ds(ebase + i * _CHUNK, _CHUNK)]

    def wait_out(i, b):
        pltpu.make_async_copy(rows.at[b, pl.ds(0, _CHUNK)], out_slice(i),
                              sem_o.at[b]).wait()

    def start_out(i, b):
        pltpu.async_copy(rows.at[b, pl.ds(0, _CHUNK)], out_slice(i),
                         sem_o.at[b])

    wait_idx(0, 0)
    pltpu.async_copy(table.at[idx_ref(0)], rows.at[0], sem_g.at[0])
    pltpu.async_copy(table.at[idx_ref(1)], rows.at[1], sem_g.at[1])
    pltpu.async_copy(table.at[idx_ref(2)], rows.at[2], sem_g.at[2])

    def body(i0, carry):
        for b2 in range(_NB):
            i = i0 * _NB + b2
            bg = (b2 + _NB - 1) % _NB
            if b2 == 0:
                @pl.when(i0 > 0)
                def _():
                    wait_out(i - 1, bg)
            else:
                wait_out(i - 1, bg)

            @pl.when(i + (_NB - 1) < _CH_PER_W)
            def _():
                issue_gather(i + (_NB - 1), bg)

            wait_gather(i, b2)
            prefetch_idx(i)
            subtract(b2)
            start_out(i, b2)
        return carry

    lax.fori_loop(0, (_CH_PER_W - 2) // _NB, body, 0)

    for i in (_CH_PER_W - 2, _CH_PER_W - 1):
        b = i % _NB
        wait_out(i - 1, (i - 1) % _NB)
        wait_gather(i, b)
        subtract(b)
        start_out(i, b)
    wait_out(_CH_PER_W - 1, (_CH_PER_W - 1) % _NB)


def _norm_body(x_ref, o_ref):
    o_ref[...] = (x_ref[...] - _STAT_MEDIAN) * _STAT_SCALE


_norm_call = pl.pallas_call(
    _norm_body,
    out_shape=jax.ShapeDtypeStruct((_N_NODES, _NODE_FEATS), jnp.float32),
)


def kernel(node_feature, edge_index):
    ei = edge_index.astype(jnp.int32)
    cidx = ei.reshape(2, _N_EDGES // _CHUNK, _CHUNK).transpose(1, 0, 2)
    cidx = cidx.reshape(-1)
    norm = _norm_call(node_feature)
    edge_feature = _edge_kernel(node_feature, cidx)
    return (norm, edge_feature)


# speedup vs baseline: 1.2950x; 1.2950x over previous
import functools

import jax
import jax.numpy as jnp
from jax import lax
from jax.experimental import pallas as pl
from jax.experimental.pallas import tpu as pltpu
from jax.experimental.pallas import tpu_sc as plsc

_NODE_FEATS = 128
_STAT_MEDIAN = 0.0
_STAT_SCALE = 2.0 / (1.0 - (-1.0))
_N_NODES = 10000
_N_EDGES = 320000

_LANES = 16
_NW = 32
_E_PER_W = _N_EDGES // _NW
_CHUNK = 80
_CH_PER_W = _E_PER_W // _CHUNK
_IDX_BLK = 25
_N_BLKS = _CH_PER_W // _IDX_BLK

_mesh = plsc.VectorSubcoreMesh(core_axis_name="c", subcore_axis_name="s")


@functools.partial(
    pl.kernel,
    mesh=_mesh,
    out_type=jax.ShapeDtypeStruct((_N_EDGES, _NODE_FEATS), jnp.float32),
    scratch_types=[
        pltpu.VMEM_SHARED((_N_NODES, _NODE_FEATS), jnp.float32),
        pltpu.VMEM((2 * _IDX_BLK * _CHUNK,), jnp.int32),
        pltpu.VMEM((2 * _IDX_BLK * _CHUNK,), jnp.int32),
        pltpu.VMEM((2, _CHUNK, _NODE_FEATS), jnp.float32),
        pltpu.VMEM((2, _CHUNK, _NODE_FEATS), jnp.float32),
        pltpu.SemaphoreType.DMA((2,)),
        pltpu.SemaphoreType.DMA((2,)),
        pltpu.SemaphoreType.DMA((2,)),
        pltpu.SemaphoreType.DMA((2,)),
    ],
)
def _edge_kernel(node_hbm, src_hbm, dst_hbm, out_hbm,
                 table, sidx, didx, srows, drows, sem_s, sem_d, sem_o, sem_i):
    wid = lax.axis_index("s") * 2 + lax.axis_index("c")
    ebase = wid * _E_PER_W
    sid = lax.axis_index("s")

    rows_per_sub = 624
    tslice = pl.ds(sid * rows_per_sub, rows_per_sub)
    pltpu.async_copy(node_hbm.at[tslice], table.at[tslice], sem_o.at[0])
    rem = pl.ds(16 * rows_per_sub, _N_NODES - 16 * rows_per_sub)

    @pl.when(sid == 0)
    def _():
        pltpu.async_copy(node_hbm.at[rem], table.at[rem], sem_o.at[1])

    _BLK_E = _IDX_BLK * _CHUNK

    def fetch_idx(j, jbuf):
        ibase = ebase + j * _BLK_E
        vsl = pl.ds(jbuf * _BLK_E, _BLK_E)
        pltpu.async_copy(src_hbm.at[pl.ds(ibase, _BLK_E)], sidx.at[vsl],
                         sem_i.at[jbuf])
        pltpu.async_copy(dst_hbm.at[pl.ds(ibase, _BLK_E)], didx.at[vsl],
                         sem_i.at[jbuf])

    def wait_idx(j, jbuf):
        ibase = ebase + j * _BLK_E
        vsl = pl.ds(jbuf * _BLK_E, _BLK_E)
        pltpu.make_async_copy(src_hbm.at[pl.ds(ibase, _BLK_E)],
                              sidx.at[vsl], sem_i.at[jbuf]).wait()
        pltpu.make_async_copy(dst_hbm.at[pl.ds(ibase, _BLK_E)],
                              didx.at[vsl], sem_i.at[jbuf]).wait()

    fetch_idx(0, 0)
    fetch_idx(1, 1)
    pltpu.make_async_copy(node_hbm.at[tslice], table.at[tslice],
                          sem_o.at[0]).wait()

    @pl.when(sid == 0)
    def _():
        pltpu.make_async_copy(node_hbm.at[rem], table.at[rem],
                              sem_o.at[1]).wait()

    plsc.subcore_barrier()

    def idx_refs(i):
        off = ((i // _IDX_BLK) % 2) * _BLK_E + (i % _IDX_BLK) * _CHUNK
        return (sidx.at[pl.ds(off, _CHUNK)], didx.at[pl.ds(off, _CHUNK)])

    def issue_gather(i, b):
        @pl.when(i % _IDX_BLK == 0)
        def _():
            wait_idx(i // _IDX_BLK, (i // _IDX_BLK) % 2)

        s_ix, d_ix = idx_refs(i)
        pltpu.async_copy(table.at[s_ix], srows.at[b], sem_s.at[b])
        pltpu.async_copy(table.at[d_ix], drows.at[b], sem_d.at[b])

    def prefetch_idx(i):
        j2 = i // _IDX_BLK + 2

        @pl.when((i % _IDX_BLK == _IDX_BLK - 1) & (j2 < _N_BLKS))
        def _():
            fetch_idx(j2, j2 % 2)

    def wait_gather(i, b):
        s_ix, d_ix = idx_refs(i)
        pltpu.make_async_copy(table.at[s_ix], srows.at[b], sem_s.at[b]).wait()
        pltpu.make_async_copy(table.at[d_ix], drows.at[b], sem_d.at[b]).wait()

    _HALF = _CHUNK // 2

    def subtract_half(b, h):
        def sub_row(r, carry2):
            for r2 in range(2):
                rr = h * _HALF + 2 * r + r2
                for q in range(_NODE_FEATS // _LANES):
                    sl = pl.ds(q * _LANES, _LANES)
                    if _STAT_SCALE == 1.0:
                        plsc.addupdate(srows.at[b, rr, sl],
                                       -drows[b, rr, sl])
                    else:
                        srows[b, rr, sl] = (
                            srows[b, rr, sl]
                            - drows[b, rr, sl]) * _STAT_SCALE
            return carry2

        lax.fori_loop(0, _HALF // 2, sub_row, 0)

    def out_half(i, b, h):
        return (srows.at[b, pl.ds(h * _HALF, _HALF)],
                out_hbm.at[pl.ds(ebase + i * _CHUNK + h * _HALF, _HALF)])

    def start_out(i, b, h):
        s, d = out_half(i, b, h)
        pltpu.async_copy(s, d, sem_o.at[b])

    def wait_out(i, b):
        for h in range(2):
            s, d = out_half(i, b, h)
            pltpu.make_async_copy(s, d, sem_o.at[b]).wait()

    wait_idx(0, 0)
    s_ix0, d_ix0 = idx_refs(0)
    pltpu.async_copy(table.at[s_ix0], srows.at[0], sem_s.at[0])
    pltpu.async_copy(table.at[d_ix0], drows.at[0], sem_d.at[0])

    def body(i0, carry):
        for b2 in range(2):
            i = i0 * 2 + b2
            bnext = 1 - b2
            if b2 == 0:
                @pl.when(i0 > 0)
                def _():
                    wait_out(i - 1, bnext)
            else:
                wait_out(i - 1, bnext)
            issue_gather(i + 1, bnext)
            wait_gather(i, b2)
            prefetch_idx(i)
            subtract_half(b2, 0)
            start_out(i, b2, 0)
            subtract_half(b2, 1)
            start_out(i, b2, 1)
        return carry

    lax.fori_loop(0, (_CH_PER_W - 1) // 2, body, 0)

    wait_out(_CH_PER_W - 2, 1)
    wait_gather(_CH_PER_W - 1, 0)
    subtract_half(0, 0)
    start_out(_CH_PER_W - 1, 0, 0)
    subtract_half(0, 1)
    start_out(_CH_PER_W - 1, 0, 1)
    wait_out(_CH_PER_W - 1, 0)


def _norm_body(x_ref, o_ref):
    o_ref[...] = (x_ref[...] - _STAT_MEDIAN) * _STAT_SCALE


_norm_call = pl.pallas_call(
    _norm_body,
    out_shape=jax.ShapeDtypeStruct((_N_NODES, _NODE_FEATS), jnp.float32),
)


def kernel(node_feature, edge_index):
    src = edge_index[0].astype(jnp.int32)
    dst = edge_index[1].astype(jnp.int32)
    norm = _norm_call(node_feature)
    edge_feature = _edge_kernel(node_feature, src, dst)
    return (norm, edge_feature)
